# baseline (speedup 1.0000x reference)
import functools

import jax
import jax.numpy as jnp
from jax import lax
from jax.experimental import pallas as pl
from jax.experimental.pallas import tpu as pltpu
from jax.experimental.pallas import tpu_sc as plsc

_ROWS = 4096
_TROWS = 12288 - _ROWS
_TBLK = 256
_HW = 1024
_K = 256
_NW = 32
_RPW = _ROWS // _NW
_CH = 16
_NCH = _RPW // _CH
_CHW = _CH * _HW
_HPAD = 257
_ABS = 0x7FFFFFFF

_LEVELS = (
    (23, 0xFF, 256, None),
    (15, 0xFF, 256, 23),
    (7, 0xFF, 256, 15),
    (0, 0x7F, 128, 7),
)


def _sc_body(x_hbm, tau_hbm, out_hbm, xs0, xs1, ov0, ov1, hist, tausc,
             sem0, sem1, semo0, semo1):
    wid = lax.axis_index("c") * 16 + lax.axis_index("s")
    lanes = lax.broadcasted_iota(jnp.int32, (16,), 0)
    hbase = lanes * _HPAD
    ones = jnp.ones((16,), jnp.int32)
    zeros16 = jnp.zeros((16,), jnp.int32)
    wbase = wid * _RPW * _HW

    pltpu.sync_copy(tau_hbm, tausc)
    tauv = tausc[...]
    tau1m = 1.0 - tauv

    def in_copy(g, buf, sem):
        pltpu.async_copy(
            x_hbm.at[pl.ds(wbase + g * _CHW, _CHW)], buf, sem)

    def in_drain(g, buf, sem):
        pltpu.make_async_copy(
            x_hbm.at[pl.ds(wbase + g * _CHW, _CHW)], buf, sem).wait()

    def out_drain(g, buf, sem):
        pltpu.make_async_copy(
            buf, out_hbm.at[pl.ds(wbase + g * _CHW, _CHW)], sem).wait()

    def process(g, xs, ovb, osem):
        kvec = jnp.full((16,), _K, jnp.int32)
        pfx = zeros16

        for shift, dmask, bins, pshift in _LEVELS:
            pfxs = (None if pshift is None else
                    [pfx[r] for r in range(_CH)])

            @plsc.parallel_loop(0, _HW // 16, unroll=2)
            def _(c0):
                col = c0 * 16
                for r in range(_CH):
                    v = xs[pl.ds(r * _HW + col, 16)]
                    au = plsc.bitcast(v, jnp.int32) & _ABS
                    d = lax.shift_right_logical(au, shift) & dmask
                    if pshift is None:
                        plsc.addupdate_scatter(hist, [d + r * _HPAD], ones)
                    else:
                        m = lax.shift_right_logical(au, pshift) == pfxs[r]
                        plsc.addupdate_scatter(hist, [d + r * _HPAD], ones,
                                               mask=m)

            def scan_body(i, carry):
                s, dig, kp = carry
                for k in range(4):
                    b = (bins - 1) - (i * 4 + k)
                    cnt = plsc.load_gather(hist, [hbase + b])
                    plsc.store_scatter(hist, [hbase + b], zeros16)
                    s_new = s + cnt
                    crossed = jnp.logical_and(s < kvec, s_new >= kvec)
                    dig = jnp.where(crossed, b, dig)
                    kp = jnp.where(crossed, kvec - s, kp)
                    s = s_new
                return (s, dig, kp)

            _, dig, kp = plsc.parallel_loop(
                0, bins // 4, unroll=4,
                carry=(zeros16, zeros16, kvec))(scan_body)
            kvec = kp
            if pshift is None:
                pfx = dig
            elif shift > 0:
                pfx = (pfx << 8) | dig
            else:
                thresh = (pfx << 7) | dig

        thrs = [thresh[r] for r in range(_CH)]

        @plsc.parallel_loop(0, _HW // 16, unroll=2)
        def _(c0):
            col = c0 * 16
            for r in range(_CH):
                v = xs[pl.ds(r * _HW + col, 16)]
                au = plsc.bitcast(v, jnp.int32) & _ABS
                sp = jnp.where(au >= thrs[r], v, jnp.float32(0.0))
                ovb[pl.ds(r * _HW + col, 16)] = sp * tauv + v * tau1m

        pltpu.async_copy(
            ovb, out_hbm.at[pl.ds(wbase + g * _CHW, _CHW)], osem)

    @plsc.parallel_loop(0, 16, unroll=2)
    def _(j):
        for k in range(16):
            hist[pl.ds(j * 256 + k * 16, 16)] = zeros16

    hist[pl.ds(4096, 16)] = zeros16

    in_copy(0, xs0, sem0)

    def pair_body(j, carry):
        g0 = j * 2
        g1 = g0 + 1
        in_drain(g0, xs0, sem0)
        in_copy(g1, xs1, sem1)

        @pl.when(j > 0)
        def _():
            out_drain(g0 - 2, ov0, semo0)

        process(g0, xs0, ov0, semo0)
        in_drain(g1, xs1, sem1)
        g2 = jnp.minimum(g1 + 1, _NCH - 1)
        in_copy(g2, xs0, sem0)

        @pl.when(j > 0)
        def _():
            out_drain(g1 - 2, ov1, semo1)

        process(g1, xs1, ov1, semo1)
        return carry

    lax.fori_loop(0, _NCH // 2, pair_body, 0)
    in_drain(_NCH - 1, xs0, sem0)
    out_drain(_NCH - 2, ov0, semo0)
    out_drain(_NCH - 1, ov1, semo1)


_sc_call = functools.partial(
    pl.kernel,
    out_type=jax.ShapeDtypeStruct((_ROWS * _HW,), jnp.float32),
    mesh=plsc.VectorSubcoreMesh(core_axis_name="c", subcore_axis_name="s"),
    scratch_types=[
        pltpu.VMEM((_CHW,), jnp.float32),
        pltpu.VMEM((_CHW,), jnp.float32),
        pltpu.VMEM((_CHW,), jnp.float32),
        pltpu.VMEM((_CHW,), jnp.float32),
        pltpu.VMEM((_CH * _HPAD,), jnp.int32),
        pltpu.VMEM((16,), jnp.float32),
        pltpu.SemaphoreType.DMA,
        pltpu.SemaphoreType.DMA,
        pltpu.SemaphoreType.DMA,
        pltpu.SemaphoreType.DMA,
    ],
    compiler_params=pltpu.CompilerParams(needs_layout_passes=False),
)(_sc_body)


def _tc_body(tau_ref, x_ref, sc_ref, o_ref):
    i = pl.program_id(0)

    @pl.when(i < _ROWS // _TBLK)
    def _():
        o_ref[...] = sc_ref[...]

    @pl.when(i >= _ROWS // _TBLK)
    def _():
        x = x_ref[...]
        u = jax.lax.bitcast_convert_type(x, jnp.int32) & jnp.int32(_ABS)
        t = jnp.zeros((x.shape[0], 1), jnp.int32)
        for b in range(30, -1, -1):
            cand = t | jnp.int32(1 << b)
            cnt = jnp.sum((u >= cand).astype(jnp.int32), axis=1,
                          keepdims=True)
            t = jnp.where(cnt >= _K, cand, t)
        sparse = jnp.where(u >= t, x, jnp.float32(0.0))
        tau = tau_ref[0]
        o_ref[...] = sparse * tau + x * (jnp.float32(1.0) - tau)


@jax.jit
def kernel(x, tau):
    n, c, h, w = x.shape
    x2 = x.reshape(n * c, h * w)
    tau_arr = jnp.full((16,), tau, jnp.float32)
    tau_s = jnp.asarray(tau, jnp.float32).reshape(1)
    out_sc = _sc_call(x2[:_ROWS].reshape(-1), tau_arr)
    nsc = _ROWS // _TBLK
    out = pl.pallas_call(
        _tc_body,
        grid=(12288 // _TBLK,),
        in_specs=[
            pl.BlockSpec(memory_space=pltpu.MemorySpace.SMEM),
            pl.BlockSpec((_TBLK, _HW), lambda i: (i, 0)),
            pl.BlockSpec((_TBLK, _HW),
                         lambda i: (jnp.minimum(i, nsc - 1), 0)),
        ],
        out_specs=pl.BlockSpec((_TBLK, _HW), lambda i: (i, 0)),
        out_shape=jax.ShapeDtypeStruct((12288, _HW), jnp.float32),
    )(tau_s, x2, out_sc.reshape(_ROWS, _HW))
    return out.reshape(n, c, h, w)


# speedup vs baseline: 1.0805x; 1.0805x over previous
import functools

import jax
import jax.numpy as jnp
from jax import lax
from jax.experimental import pallas as pl
from jax.experimental.pallas import tpu as pltpu
from jax.experimental.pallas import tpu_sc as plsc

_ROWS = 4096
_TROWS = 12288 - _ROWS
_TBLK = 256
_HW = 1024
_K = 256
_NW = 32
_RPW = _ROWS // _NW
_CH = 16
_NCH = _RPW // _CH
_CHW = _CH * _HW
_HPAD = 257
_ABS = 0x7FFFFFFF

_LEVELS = (
    (23, 0xFF, 256, None),
    (15, 0xFF, 256, 23),
    (7, 0xFF, 256, 15),
    (0, 0x7F, 128, 7),
)


def _sc_body(x_hbm, tau_hbm, out_hbm, xs0, xs1, ov0, ov1, hist, tausc,
             sem0, sem1, semo0, semo1):
    wid = lax.axis_index("c") * 16 + lax.axis_index("s")
    lanes = lax.broadcasted_iota(jnp.int32, (16,), 0)
    hbase = lanes * _HPAD
    ones = jnp.ones((16,), jnp.int32)
    zeros16 = jnp.zeros((16,), jnp.int32)
    wbase = wid * _RPW * _HW

    pltpu.sync_copy(tau_hbm, tausc)
    tauv = tausc[...]
    tau1m = 1.0 - tauv

    def in_copy(g, buf, sem):
        pltpu.async_copy(
            x_hbm.at[pl.ds(wbase + g * _CHW, _CHW)], buf, sem)

    def in_drain(g, buf, sem):
        pltpu.make_async_copy(
            x_hbm.at[pl.ds(wbase + g * _CHW, _CHW)], buf, sem).wait()

    def out_drain(g, buf, sem):
        pltpu.make_async_copy(
            buf, out_hbm.at[pl.ds(wbase + g * _CHW, _CHW)], sem).wait()

    def process(g, xs, ovb, osem):
        kvec = jnp.full((16,), _K, jnp.int32)
        pfx = zeros16

        for shift, dmask, bins, pshift in _LEVELS:
            pfxs = (None if pshift is None else
                    [pfx[r] for r in range(_CH)])

            @plsc.parallel_loop(0, _HW // 16, unroll=2)
            def _(c0):
                col = c0 * 16
                for r in range(_CH):
                    v = xs[pl.ds(r * _HW + col, 16)]
                    au = plsc.bitcast(v, jnp.int32) & _ABS
                    d = lax.shift_right_logical(au, shift) & dmask
                    if pshift is None:
                        plsc.addupdate_scatter(hist, [d + r * _HPAD], ones)
                    else:
                        m = lax.shift_right_logical(au, pshift) == pfxs[r]
                        plsc.addupdate_scatter(hist, [d + r * _HPAD], ones,
                                               mask=m)

            def scan_body(i, carry):
                s, dig, kp = carry
                for k in range(4):
                    b = (bins - 1) - (i * 4 + k)
                    cnt = plsc.load_gather(hist, [hbase + b])
                    plsc.store_scatter(hist, [hbase + b], zeros16)
                    s_new = s + cnt
                    crossed = jnp.logical_and(s < kvec, s_new >= kvec)
                    dig = jnp.where(crossed, b, dig)
                    kp = jnp.where(crossed, kvec - s, kp)
                    s = s_new
                return (s, dig, kp)

            _, dig, kp = plsc.parallel_loop(
                0, bins // 4, unroll=4,
                carry=(zeros16, zeros16, kvec))(scan_body)
            kvec = kp
            if pshift is None:
                pfx = dig
            elif shift > 0:
                pfx = (pfx << 8) | dig
            else:
                thresh = (pfx << 7) | dig

        thrs = [thresh[r] for r in range(_CH)]

        @plsc.parallel_loop(0, _HW // 16, unroll=2)
        def _(c0):
            col = c0 * 16
            for r in range(_CH):
                v = xs[pl.ds(r * _HW + col, 16)]
                au = plsc.bitcast(v, jnp.int32) & _ABS
                sp = jnp.where(au >= thrs[r], v, jnp.float32(0.0))
                ovb[pl.ds(r * _HW + col, 16)] = sp * tauv + v * tau1m

        pltpu.async_copy(
            ovb, out_hbm.at[pl.ds(wbase + g * _CHW, _CHW)], osem)

    @plsc.parallel_loop(0, 16, unroll=2)
    def _(j):
        for k in range(16):
            hist[pl.ds(j * 256 + k * 16, 16)] = zeros16

    hist[pl.ds(4096, 16)] = zeros16

    in_copy(0, xs0, sem0)

    def pair_body(j, carry):
        g0 = j * 2
        g1 = g0 + 1
        in_drain(g0, xs0, sem0)
        in_copy(g1, xs1, sem1)

        @pl.when(j > 0)
        def _():
            out_drain(g0 - 2, ov0, semo0)

        process(g0, xs0, ov0, semo0)
        in_drain(g1, xs1, sem1)
        g2 = jnp.minimum(g1 + 1, _NCH - 1)
        in_copy(g2, xs0, sem0)

        @pl.when(j > 0)
        def _():
            out_drain(g1 - 2, ov1, semo1)

        process(g1, xs1, ov1, semo1)
        return carry

    lax.fori_loop(0, _NCH // 2, pair_body, 0)
    in_drain(_NCH - 1, xs0, sem0)
    out_drain(_NCH - 2, ov0, semo0)
    out_drain(_NCH - 1, ov1, semo1)


_sc_call = functools.partial(
    pl.kernel,
    out_type=jax.ShapeDtypeStruct((_ROWS * _HW,), jnp.float32),
    mesh=plsc.VectorSubcoreMesh(core_axis_name="c", subcore_axis_name="s"),
    scratch_types=[
        pltpu.VMEM((_CHW,), jnp.float32),
        pltpu.VMEM((_CHW,), jnp.float32),
        pltpu.VMEM((_CHW,), jnp.float32),
        pltpu.VMEM((_CHW,), jnp.float32),
        pltpu.VMEM((_CH * _HPAD,), jnp.int32),
        pltpu.VMEM((16,), jnp.float32),
        pltpu.SemaphoreType.DMA,
        pltpu.SemaphoreType.DMA,
        pltpu.SemaphoreType.DMA,
        pltpu.SemaphoreType.DMA,
    ],
    compiler_params=pltpu.CompilerParams(needs_layout_passes=False),
)(_sc_body)


def _tc_body(tau_ref, x_ref, o_ref):
    x = x_ref[...]
    u = jax.lax.bitcast_convert_type(x, jnp.int32) & jnp.int32(_ABS)
    t = jnp.zeros((x.shape[0], 1), jnp.int32)
    for b in range(30, -1, -1):
        cand = t | jnp.int32(1 << b)
        cnt = jnp.sum((u >= cand).astype(jnp.int32), axis=1, keepdims=True)
        t = jnp.where(cnt >= _K, cand, t)
    sparse = jnp.where(u >= t, x, jnp.float32(0.0))
    tau = tau_ref[0]
    o_ref[...] = sparse * tau + x * (jnp.float32(1.0) - tau)


@jax.jit
def kernel(x, tau):
    n, c, h, w = x.shape
    x2 = x.reshape(n * c, h * w)
    tau_arr = jnp.full((16,), tau, jnp.float32)
    tau_s = jnp.asarray(tau, jnp.float32).reshape(1)
    out_sc = _sc_call(x2[:_ROWS].reshape(-1), tau_arr)
    out_tc = pl.pallas_call(
        _tc_body,
        grid=(_TROWS // _TBLK,),
        in_specs=[
            pl.BlockSpec(memory_space=pltpu.MemorySpace.SMEM),
            pl.BlockSpec((_TBLK, _HW), lambda i: (i + _ROWS // _TBLK, 0)),
        ],
        out_specs=pl.BlockSpec((_TBLK, _HW), lambda i: (i, 0)),
        out_shape=jax.ShapeDtypeStruct((_TROWS, _HW), jnp.float32),
    )(tau_s, x2)
    out = jnp.concatenate([out_sc.reshape(_ROWS, _HW), out_tc], axis=0)
    return out.reshape(n, c, h, w)
